# baseline (speedup 1.0000x reference)
import functools

import jax
import jax.numpy as jnp
import numpy as np
from jax import lax
from jax.experimental import pallas as pl
from jax.experimental.pallas import tpu as pltpu
from jax.experimental.pallas import tpu_sc as plsc

_B, _H, _W, _N = 8, 512, 512, 128
_HW = _H * _W
_COUNT_WEIGHT = 2.0
_SPATIAL_WEIGHT = 0.15

_OFFS = [(dy, dx) for dy in (-1, 0, 1) for dx in (-1, 0, 1)]
_WTS = [float(np.exp(-np.sqrt(dx * dx + dy * dy) / 2.0)) for dy, dx in _OFFS]

_NC, _NS = 2, 16
_IMGS_PER_CORE = _B // _NC
_TILES_PER_IMG = _NS // _IMGS_PER_CORE
_PTS_PER_TILE = _N // _TILES_PER_IMG
_QTR = _HW // _TILES_PER_IMG
_ROWS_PER_TILE = _H // _TILES_PER_IMG
_NCON = _PTS_PER_TILE * 9
_ROWS, _COLS = 3, 96

_UNROLL = 8
_INNER = _QTR // (16 * _UNROLL)


def _cell_off(y, x):
    return y * _W + x


def _sc_body(pred_hbm, px_hbm, py_hbm, ca_hbm, out_hbm,
             grid_sh, stage_sh, pidx_sh, pw_sh,
             xs_v, ys_v, idx_v, w_v, t_v, z_v,
             post_idx_v, post_w_v, sib_idx_v, sib_w_v,
             dense_v, vec_v, ca_v, dsem):
    c = lax.axis_index("c")
    s = lax.axis_index("s")
    slot = s // _TILES_PER_IMG
    q = s % _TILES_PER_IMG
    img = c * _IMGS_PER_CORE + slot
    lanes = lax.iota(jnp.int32, 16)

    dense_cp = pltpu.async_copy(
        pred_hbm.at[pl.ds(img * _H + q * _ROWS_PER_TILE, _ROWS_PER_TILE), :],
        dense_v, dsem)

    pltpu.sync_copy(px_hbm.at[img, pl.ds(q * _PTS_PER_TILE, _PTS_PER_TILE)], xs_v)
    pltpu.sync_copy(py_hbm.at[img, pl.ds(q * _PTS_PER_TILE, _PTS_PER_TILE)], ys_v)

    t_acc = jnp.zeros((16,), jnp.float32)
    for v in range(_PTS_PER_TILE // 16):
        x0 = jnp.clip(xs_v[pl.ds(v * 16, 16)], 0, _W - 1)
        y0 = jnp.clip(ys_v[pl.ds(v * 16, 16)], 0, _H - 1)
        for k, (dy, dx) in enumerate(_OFFS):
            ny = y0 + dy
            nx = x0 + dx
            valid = (ny >= 0) & (ny < _H) & (nx >= 0) & (nx < _W)
            off = _cell_off(jnp.clip(ny, 0, _H - 1), jnp.clip(nx, 0, _W - 1))
            w = jnp.where(valid, jnp.float32(_WTS[k]), jnp.float32(0.0))
            j = v * 9 + k
            r, col = j // 6, (j % 6) * 16
            idx_v[r, pl.ds(col, 16)] = off + (slot % 2) * _HW
            w_v[r, pl.ds(col, 16)] = w
            post_idx_v[pl.ds(j * 16, 16)] = off
            post_w_v[pl.ds(j * 16, 16)] = w
            t_acc = t_acc + w
    t_part = jnp.sum(t_acc)

    for u in range(_COLS // 16):
        z_v[pl.ds(u * 16, 16)] = jnp.zeros((16,), jnp.float32)

    pltpu.sync_copy(post_idx_v, pidx_sh.at[pl.ds(s * _NCON, _NCON)])
    pltpu.sync_copy(post_w_v, pw_sh.at[pl.ds(s * _NCON, _NCON)])

    for p in range(2):
        active = (slot // 2) == p

        @pl.when(active)
        def _():
            for r in range(_ROWS):
                pltpu.sync_copy(z_v, grid_sh.at[idx_v.at[r]])
        plsc.subcore_barrier()

        @pl.when(active)
        def _():
            for r in range(_ROWS):
                pltpu.sync_copy(w_v.at[r], grid_sh.at[idx_v.at[r]], add=True)
        plsc.subcore_barrier()

        @pl.when(active)
        def _():
            for r in range(_ROWS):
                pltpu.sync_copy(grid_sh.at[idx_v.at[r]], t_v.at[r])
        plsc.subcore_barrier()

    u_acc = jnp.zeros((16,), jnp.float32)
    for r in range(_ROWS):
        for u in range(_COLS // 16):
            u_acc = u_acc + (w_v[r, pl.ds(u * 16, 16)]
                             * t_v[r, pl.ds(u * 16, 16)])
    u_part = jnp.sum(u_acc)

    dense_cp.wait()

    zero16 = jnp.zeros((16,), jnp.float32)

    @plsc.parallel_loop(0, _ROWS_PER_TILE, 1, unroll=2,
                        carry=tuple([zero16] * (2 * _UNROLL)))
    def accs(row, carry):
        accs = list(carry)
        for cblk in range(4):
            for u in range(_UNROLL):
                vv = dense_v[row, pl.ds(cblk * 128 + u * 16, 16)]
                accs[u] = accs[u] + vv
                accs[_UNROLL + u] = accs[_UNROLL + u] + vv * vv
        return tuple(accs)
    s_vec = accs[0]
    q_vec = accs[_UNROLL]
    for u in range(1, _UNROLL):
        s_vec = s_vec + accs[u]
        q_vec = q_vec + accs[_UNROLL + u]
    s_part = jnp.sum(s_vec)
    q_part = jnp.sum(q_vec)

    lo = q * _QTR
    g_acc = jnp.zeros((16,), jnp.float32)
    pltpu.sync_copy(
        pidx_sh.at[pl.ds((s - q) * _NCON, _TILES_PER_IMG * _NCON)], sib_idx_v)
    pltpu.sync_copy(
        pw_sh.at[pl.ds((s - q) * _NCON, _TILES_PER_IMG * _NCON)], sib_w_v)
    for u in range(_TILES_PER_IMG * _NCON // 16):
        off = sib_idx_v[pl.ds(u * 16, 16)]
        wv = sib_w_v[pl.ds(u * 16, 16)]
        inb = (off >= lo) & (off < lo + _QTR)
        loc = jnp.clip(off - lo, 0, _QTR - 1)
        val = plsc.load_gather(dense_v, [loc >> 9, loc & 511])
        g_acc = g_acc + jnp.where(inb, wv, jnp.float32(0.0)) * val
    g_part = jnp.sum(g_acc)

    part = jnp.where(lanes == 0, s_part,
           jnp.where(lanes == 1, q_part,
           jnp.where(lanes == 2, t_part,
           jnp.where(lanes == 3, g_part,
           jnp.where(lanes == 4, u_part, jnp.float32(0.0))))))
    vec_v[...] = part
    pltpu.sync_copy(vec_v, stage_sh.at[pl.ds(s * 16, 16)])
    plsc.subcore_barrier()

    @pl.when(s == 0)
    def _():
        pltpu.sync_copy(ca_hbm, ca_v)
        ca_vec = ca_v[...]
        zero16f = jnp.zeros((16,), jnp.float32)
        loss_vec = zero16f
        pltpu.sync_copy(stage_sh, sib_w_v.at[pl.ds(0, _NS * 16)])
        for i_loc in range(_IMGS_PER_CORE):
            b = 4 * i_loc * 16
            acc = (sib_w_v[pl.ds(b, 16)] + sib_w_v[pl.ds(b + 16, 16)]
                   + sib_w_v[pl.ds(b + 32, 16)] + sib_w_v[pl.ds(b + 48, 16)])
            S = zero16f + jnp.sum(jnp.where(lanes == 0, acc, jnp.float32(0.0)))
            Q = zero16f + jnp.sum(jnp.where(lanes == 1, acc, jnp.float32(0.0)))
            T = zero16f + jnp.sum(jnp.where(lanes == 2, acc, jnp.float32(0.0)))
            G = zero16f + jnp.sum(jnp.where(lanes == 3, acc, jnp.float32(0.0)))
            U = zero16f + jnp.sum(jnp.where(lanes == 4, acc, jnp.float32(0.0)))
            D = S + jnp.float32(1e-8)
            count = jnp.abs(S / ca_vec - jnp.float32(_N))
            spatial = (Q / (D * D) - 2.0 * G / (D * T) + U / (T * T)) * jnp.float32(1.0 / _HW)
            loss_vec = (loss_vec + jnp.float32(_COUNT_WEIGHT / _B) * count
                        + jnp.float32(_SPATIAL_WEIGHT / _B) * spatial)
        vec_v[...] = jnp.where(lanes == 0, loss_vec, zero16f)
        pltpu.sync_copy(vec_v, out_hbm.at[c])


@jax.jit
def _loss_sc(pred2d, px, py, ca_vec):
    mesh = plsc.VectorSubcoreMesh(core_axis_name="c", subcore_axis_name="s")
    fn = pl.kernel(
        _sc_body,
        mesh=mesh,
        compiler_params=pltpu.CompilerParams(
            needs_layout_passes=False, use_tc_tiling_on_sc=True),
        out_type=jax.ShapeDtypeStruct((_NC, 16), jnp.float32),
        scratch_types=[
            pltpu.MemorySpace.VMEM_SHARED((2 * _HW,), jnp.float32),
            pltpu.MemorySpace.VMEM_SHARED((_NS * 16,), jnp.float32),
            pltpu.MemorySpace.VMEM_SHARED((_NS * _NCON,), jnp.int32),
            pltpu.MemorySpace.VMEM_SHARED((_NS * _NCON,), jnp.float32),
            pltpu.MemorySpace.VMEM((_PTS_PER_TILE,), jnp.int32),
            pltpu.MemorySpace.VMEM((_PTS_PER_TILE,), jnp.int32),
            pltpu.MemorySpace.VMEM((_ROWS, _COLS), jnp.int32),
            pltpu.MemorySpace.VMEM((_ROWS, _COLS), jnp.float32),
            pltpu.MemorySpace.VMEM((_ROWS, _COLS), jnp.float32),
            pltpu.MemorySpace.VMEM((_COLS,), jnp.float32),
            pltpu.MemorySpace.VMEM((_NCON,), jnp.int32),
            pltpu.MemorySpace.VMEM((_NCON,), jnp.float32),
            pltpu.MemorySpace.VMEM((_TILES_PER_IMG * _NCON,), jnp.int32),
            pltpu.MemorySpace.VMEM((_TILES_PER_IMG * _NCON,), jnp.float32),
            pltpu.MemorySpace.VMEM((_ROWS_PER_TILE, _W), jnp.float32),
            pltpu.MemorySpace.VMEM((16,), jnp.float32),
            pltpu.MemorySpace.VMEM((16,), jnp.float32),
            pltpu.SemaphoreType.DMA,
        ],
    )
    return fn(pred2d, px, py, ca_vec)


def kernel(pred_density, points_list, cell_area):
    pred2d = pred_density.reshape(_B * _H, _W)
    px = points_list[..., 0]
    py = points_list[..., 1]
    ca_vec = jnp.full((16,), cell_area, jnp.float32)
    out = _loss_sc(pred2d, px, py, ca_vec)
    return jnp.sum(out)


# speedup vs baseline: 1.0272x; 1.0272x over previous
import functools

import jax
import jax.numpy as jnp
import numpy as np
from jax import lax
from jax.experimental import pallas as pl
from jax.experimental.pallas import tpu as pltpu
from jax.experimental.pallas import tpu_sc as plsc

_B, _H, _W, _N = 8, 512, 512, 128
_HW = _H * _W
_COUNT_WEIGHT = 2.0
_SPATIAL_WEIGHT = 0.15

_OFFS = [(dy, dx) for dy in (-1, 0, 1) for dx in (-1, 0, 1)]
_WTS = [float(np.exp(-np.sqrt(dx * dx + dy * dy) / 2.0)) for dy, dx in _OFFS]

_NC, _NS = 2, 16
_IMGS_PER_CORE = _B // _NC
_TILES_PER_IMG = _NS // _IMGS_PER_CORE
_PTS_PER_TILE = _N // _TILES_PER_IMG
_QTR = _HW // _TILES_PER_IMG
_ROWS_PER_TILE = _H // _TILES_PER_IMG
_NCON = _PTS_PER_TILE * 9
_ROWS, _COLS = 3, 96

_UNROLL = 8
_INNER = _QTR // (16 * _UNROLL)


def _cell_off(y, x):
    return y * _W + x


def _sc_body(pred_hbm, px_hbm, py_hbm, ca_hbm, out_hbm,
             grid_sh, stage_sh, pidx_sh, pw_sh,
             xs_v, ys_v, idx_v, w_v, t_v, z_v,
             post_idx_v, post_w_v, sib_idx_v, sib_w_v,
             dense_v, vec_v, ca_v, dsem, ssem):
    c = lax.axis_index("c")
    s = lax.axis_index("s")
    slot = s // _TILES_PER_IMG
    q = s % _TILES_PER_IMG
    img = c * _IMGS_PER_CORE + slot
    lanes = lax.iota(jnp.int32, 16)

    dense_cp = pltpu.async_copy(
        pred_hbm.at[pl.ds(img * _H + q * _ROWS_PER_TILE, _ROWS_PER_TILE), :],
        dense_v, dsem)

    cp_x = pltpu.async_copy(
        px_hbm.at[img, pl.ds(q * _PTS_PER_TILE, _PTS_PER_TILE)], xs_v, ssem)
    cp_y = pltpu.async_copy(
        py_hbm.at[img, pl.ds(q * _PTS_PER_TILE, _PTS_PER_TILE)], ys_v, ssem)
    cp_x.wait()
    cp_y.wait()

    @pl.when(s == 0)
    def _():
        pltpu.sync_copy(ca_hbm, ca_v)

    t_acc = jnp.zeros((16,), jnp.float32)
    for v in range(_PTS_PER_TILE // 16):
        x0 = jnp.clip(xs_v[pl.ds(v * 16, 16)], 0, _W - 1)
        y0 = jnp.clip(ys_v[pl.ds(v * 16, 16)], 0, _H - 1)
        for k, (dy, dx) in enumerate(_OFFS):
            ny = y0 + dy
            nx = x0 + dx
            valid = (ny >= 0) & (ny < _H) & (nx >= 0) & (nx < _W)
            off = _cell_off(jnp.clip(ny, 0, _H - 1), jnp.clip(nx, 0, _W - 1))
            w = jnp.where(valid, jnp.float32(_WTS[k]), jnp.float32(0.0))
            j = v * 9 + k
            r, col = j // 6, (j % 6) * 16
            idx_v[r, pl.ds(col, 16)] = off + (slot % 2) * _HW
            w_v[r, pl.ds(col, 16)] = w
            post_idx_v[pl.ds(j * 16, 16)] = off
            post_w_v[pl.ds(j * 16, 16)] = w
            t_acc = t_acc + w
    t_part = jnp.sum(t_acc)

    for u in range(_COLS // 16):
        z_v[pl.ds(u * 16, 16)] = jnp.zeros((16,), jnp.float32)

    cp_pi = pltpu.async_copy(post_idx_v, pidx_sh.at[pl.ds(s * _NCON, _NCON)], ssem)
    cp_pw = pltpu.async_copy(post_w_v, pw_sh.at[pl.ds(s * _NCON, _NCON)], ssem)
    cp_pi.wait()
    cp_pw.wait()

    for p in range(2):
        active = (slot // 2) == p

        @pl.when(active)
        def _():
            cps = [pltpu.async_copy(z_v, grid_sh.at[idx_v.at[r]], ssem)
                   for r in range(_ROWS)]
            for cp in cps:
                cp.wait()
        plsc.subcore_barrier()

        @pl.when(active)
        def _():
            cps = [pltpu.async_copy(w_v.at[r], grid_sh.at[idx_v.at[r]], ssem,
                                    add=True)
                   for r in range(_ROWS)]
            for cp in cps:
                cp.wait()
        plsc.subcore_barrier()

        @pl.when(active)
        def _():
            cps = [pltpu.async_copy(grid_sh.at[idx_v.at[r]], t_v.at[r], ssem)
                   for r in range(_ROWS)]
            for cp in cps:
                cp.wait()
        plsc.subcore_barrier()

    u_acc = jnp.zeros((16,), jnp.float32)
    for r in range(_ROWS):
        for u in range(_COLS // 16):
            u_acc = u_acc + (w_v[r, pl.ds(u * 16, 16)]
                             * t_v[r, pl.ds(u * 16, 16)])
    u_part = jnp.sum(u_acc)

    dense_cp.wait()

    zero16 = jnp.zeros((16,), jnp.float32)

    def row_body(row, carry):
        accs = list(carry)
        for cblk in range(4):
            for u in range(_UNROLL):
                vv = dense_v[row, pl.ds(cblk * 128 + u * 16, 16)]
                accs[u] = accs[u] + vv
                accs[_UNROLL + u] = accs[_UNROLL + u] + vv * vv
        return tuple(accs)

    accs = lax.fori_loop(0, _ROWS_PER_TILE, row_body,
                         tuple([zero16] * (2 * _UNROLL)))
    s_vec = accs[0]
    q_vec = accs[_UNROLL]
    for u in range(1, _UNROLL):
        s_vec = s_vec + accs[u]
        q_vec = q_vec + accs[_UNROLL + u]
    s_part = jnp.sum(s_vec)
    q_part = jnp.sum(q_vec)

    lo = q * _QTR
    g_acc = jnp.zeros((16,), jnp.float32)
    pltpu.sync_copy(
        pidx_sh.at[pl.ds((s - q) * _NCON, _TILES_PER_IMG * _NCON)], sib_idx_v)
    pltpu.sync_copy(
        pw_sh.at[pl.ds((s - q) * _NCON, _TILES_PER_IMG * _NCON)], sib_w_v)
    for u in range(_TILES_PER_IMG * _NCON // 16):
        off = sib_idx_v[pl.ds(u * 16, 16)]
        wv = sib_w_v[pl.ds(u * 16, 16)]
        inb = (off >= lo) & (off < lo + _QTR)
        loc = jnp.clip(off - lo, 0, _QTR - 1)
        val = plsc.load_gather(dense_v, [loc >> 9, loc & 511])
        g_acc = g_acc + jnp.where(inb, wv, jnp.float32(0.0)) * val
    g_part = jnp.sum(g_acc)

    part = jnp.where(lanes == 0, s_part,
           jnp.where(lanes == 1, q_part,
           jnp.where(lanes == 2, t_part,
           jnp.where(lanes == 3, g_part,
           jnp.where(lanes == 4, u_part, jnp.float32(0.0))))))
    vec_v[...] = part
    pltpu.sync_copy(vec_v, stage_sh.at[pl.ds(s * 16, 16)])
    plsc.subcore_barrier()

    @pl.when(s == 0)
    def _():
        ca_vec = ca_v[...]
        zero16f = jnp.zeros((16,), jnp.float32)
        loss_vec = zero16f
        pltpu.sync_copy(stage_sh, sib_w_v.at[pl.ds(0, _NS * 16)])
        for i_loc in range(_IMGS_PER_CORE):
            b = 4 * i_loc * 16
            acc = (sib_w_v[pl.ds(b, 16)] + sib_w_v[pl.ds(b + 16, 16)]
                   + sib_w_v[pl.ds(b + 32, 16)] + sib_w_v[pl.ds(b + 48, 16)])
            S = zero16f + jnp.sum(jnp.where(lanes == 0, acc, jnp.float32(0.0)))
            Q = zero16f + jnp.sum(jnp.where(lanes == 1, acc, jnp.float32(0.0)))
            T = zero16f + jnp.sum(jnp.where(lanes == 2, acc, jnp.float32(0.0)))
            G = zero16f + jnp.sum(jnp.where(lanes == 3, acc, jnp.float32(0.0)))
            U = zero16f + jnp.sum(jnp.where(lanes == 4, acc, jnp.float32(0.0)))
            D = S + jnp.float32(1e-8)
            count = jnp.abs(S / ca_vec - jnp.float32(_N))
            spatial = (Q / (D * D) - 2.0 * G / (D * T) + U / (T * T)) * jnp.float32(1.0 / _HW)
            loss_vec = (loss_vec + jnp.float32(_COUNT_WEIGHT / _B) * count
                        + jnp.float32(_SPATIAL_WEIGHT / _B) * spatial)
        vec_v[...] = jnp.where(lanes == 0, loss_vec, zero16f)
        pltpu.sync_copy(vec_v, out_hbm.at[c])


@jax.jit
def _loss_sc(pred2d, px, py, ca_vec):
    mesh = plsc.VectorSubcoreMesh(core_axis_name="c", subcore_axis_name="s")
    fn = pl.kernel(
        _sc_body,
        mesh=mesh,
        compiler_params=pltpu.CompilerParams(
            needs_layout_passes=False, use_tc_tiling_on_sc=True),
        out_type=jax.ShapeDtypeStruct((_NC, 16), jnp.float32),
        scratch_types=[
            pltpu.MemorySpace.VMEM_SHARED((2 * _HW,), jnp.float32),
            pltpu.MemorySpace.VMEM_SHARED((_NS * 16,), jnp.float32),
            pltpu.MemorySpace.VMEM_SHARED((_NS * _NCON,), jnp.int32),
            pltpu.MemorySpace.VMEM_SHARED((_NS * _NCON,), jnp.float32),
            pltpu.MemorySpace.VMEM((_PTS_PER_TILE,), jnp.int32),
            pltpu.MemorySpace.VMEM((_PTS_PER_TILE,), jnp.int32),
            pltpu.MemorySpace.VMEM((_ROWS, _COLS), jnp.int32),
            pltpu.MemorySpace.VMEM((_ROWS, _COLS), jnp.float32),
            pltpu.MemorySpace.VMEM((_ROWS, _COLS), jnp.float32),
            pltpu.MemorySpace.VMEM((_COLS,), jnp.float32),
            pltpu.MemorySpace.VMEM((_NCON,), jnp.int32),
            pltpu.MemorySpace.VMEM((_NCON,), jnp.float32),
            pltpu.MemorySpace.VMEM((_TILES_PER_IMG * _NCON,), jnp.int32),
            pltpu.MemorySpace.VMEM((_TILES_PER_IMG * _NCON,), jnp.float32),
            pltpu.MemorySpace.VMEM((_ROWS_PER_TILE, _W), jnp.float32),
            pltpu.MemorySpace.VMEM((16,), jnp.float32),
            pltpu.MemorySpace.VMEM((16,), jnp.float32),
            pltpu.SemaphoreType.DMA,
            pltpu.SemaphoreType.DMA,
        ],
    )
    return fn(pred2d, px, py, ca_vec)


def kernel(pred_density, points_list, cell_area):
    pred2d = pred_density.reshape(_B * _H, _W)
    px = points_list[..., 0]
    py = points_list[..., 1]
    ca_vec = jnp.full((16,), cell_area, jnp.float32)
    out = _loss_sc(pred2d, px, py, ca_vec)
    return jnp.sum(out)
